# baseline (speedup 1.0000x reference)
import functools

import jax
import jax.numpy as jnp
from jax import lax
from jax.experimental import pallas as pl
from jax.experimental.pallas import tpu as pltpu
from jax.experimental.pallas import tpu_sc as plsc

N = 10000
NPAD = 10240
D = 128
E = 320000
NC = 2
NS = 16
L = 16
NW = NC * NS
EPW = E // NW
K = 40
NCHUNK = EPW // K
RPT = NPAD // NS
ZR = 32
DEGW = 128

_mesh = plsc.VectorSubcoreMesh(core_axis_name="c", subcore_axis_name="s")


NBUF = 5
GRP = NCHUNK // NBUF
ZRW = 64


@functools.partial(
    pl.kernel,
    out_type=jax.ShapeDtypeStruct((NC, NPAD, DEGW), jnp.float32),
    mesh=_mesh,
    scratch_types=[
        pltpu.VMEM_SHARED((NPAD, DEGW), jnp.float32),
        pltpu.VMEM((NCHUNK, K), jnp.int32),
        pltpu.VMEM((K, DEGW), jnp.float32),
        pltpu.SemaphoreType.DMA,
        pltpu.SemaphoreType.DMA,
        pltpu.SemaphoreType.DMA,
    ],
)
def _deg_kernel(dst_hbm, zeros_hbm, ones_hbm, out_hbm, acc, didx, ones_v, isem, zsem, ssem):
    cid = lax.axis_index("c")
    sid = lax.axis_index("s")
    wid = cid * NS + sid
    rbase = sid * RPT

    idx_cp = pltpu.async_copy(dst_hbm.at[wid], didx, isem)
    one_cp = pltpu.async_copy(ones_hbm, ones_v, isem)
    zero_cp = pltpu.async_copy(
        zeros_hbm.at[pl.ds(rbase, RPT), :], acc.at[pl.ds(rbase, RPT), :], zsem)
    idx_cp.wait()
    one_cp.wait()
    zero_cp.wait()
    plsc.subcore_barrier()

    @pl.loop(0, NCHUNK // 10)
    def _grp(g):
        cps = [
            pltpu.async_copy(ones_v, acc.at[didx.at[g * 10 + b]], ssem, add=True)
            for b in range(10)
        ]
        for cp in cps:
            cp.wait()

    plsc.subcore_barrier()

    @pl.loop(0, RPT // ZRW)
    def _write(r):
        row = rbase + r * ZRW
        pltpu.sync_copy(acc.at[pl.ds(row, ZRW), :], out_hbm.at[cid, pl.ds(row, ZRW), :])


ISLOT = 2 * NBUF


@functools.partial(
    pl.kernel,
    out_type=jax.ShapeDtypeStruct((NC, NPAD, D), jnp.float32),
    mesh=_mesh,
    scratch_types=[
        pltpu.VMEM_SHARED((NPAD, D), jnp.float32),
        [pltpu.VMEM((2, K), jnp.int32)] * ISLOT,
        [pltpu.VMEM((K, D), jnp.float32)] * NBUF,
        pltpu.SemaphoreType.DMA,
        [pltpu.SemaphoreType.DMA] * ISLOT,
        [pltpu.SemaphoreType.DMA] * NBUF,
    ],
)
def _agg_kernel(table_hbm, sd_hbm, zeros_hbm, out_hbm,
                acc, sdr, rows, zsem, isem, gsem):
    cid = lax.axis_index("c")
    sid = lax.axis_index("s")
    wid = cid * NS + sid
    rbase = sid * RPT

    zero_cp = pltpu.async_copy(
        zeros_hbm.at[pl.ds(rbase, RPT), :], acc.at[pl.ds(rbase, RPT), :], zsem)

    for s in range(ISLOT):
        pltpu.async_copy(sd_hbm.at[wid, s], sdr[s], isem[s])

    for u in range(NBUF):
        pltpu.make_async_copy(sd_hbm.at[0, 0], sdr[u], isem[u]).wait()
        pltpu.async_copy(table_hbm.at[sdr[u].at[0]], rows[u], gsem[u])

    zero_cp.wait()
    plsc.subcore_barrier()

    @pl.loop(0, NCHUNK // ISLOT)
    def _grp(g):
        for u in range(ISLOT):
            ci = g * ISLOT + u
            b = u % NBUF
            s2 = (u + NBUF) % ISLOT
            pltpu.make_async_copy(table_hbm.at[sdr[0].at[0]], rows[b], gsem[b]).wait()
            pltpu.sync_copy(rows[b], acc.at[sdr[u].at[1]], add=True)

            @pl.when(ci + ISLOT < NCHUNK)
            def _():
                pltpu.async_copy(sd_hbm.at[wid, ci + ISLOT], sdr[u], isem[u])

            @pl.when(ci + NBUF < NCHUNK)
            def _():
                pltpu.make_async_copy(sd_hbm.at[0, 0], sdr[s2], isem[s2]).wait()
                pltpu.async_copy(table_hbm.at[sdr[s2].at[0]], rows[b], gsem[b])

    plsc.subcore_barrier()

    @pl.loop(0, RPT // ZRW)
    def _write(r):
        row = rbase + r * ZRW
        pltpu.sync_copy(acc.at[pl.ds(row, ZRW), :], out_hbm.at[cid, pl.ds(row, ZRW), :])


BLK = 2000


def _dinv_block(dega, degb):
    deg = jnp.max(dega, axis=1, keepdims=True) + jnp.max(degb, axis=1, keepdims=True) + 1.0
    return lax.rsqrt(deg)


def _deg_specs():
    return [
        pl.BlockSpec((1, BLK, DEGW), lambda i: (0, i, 0)),
        pl.BlockSpec((1, BLK, DEGW), lambda i: (1, i, 0)),
    ]


def _scale_matmul_body(dega_ref, degb_ref, x_ref, w_ref, hs_ref):
    dinv = _dinv_block(dega_ref[0], degb_ref[0])
    h = jnp.dot(x_ref[...], w_ref[...], preferred_element_type=jnp.float32)
    hs_ref[...] = h * dinv


def _tc_scale_matmul(degp, x, w):
    return pl.pallas_call(
        _scale_matmul_body,
        grid=(N // BLK,),
        in_specs=_deg_specs() + [
            pl.BlockSpec((BLK, D), lambda i: (i, 0)),
            pl.BlockSpec((D, D), lambda i: (0, 0)),
        ],
        out_specs=pl.BlockSpec((BLK, D), lambda i: (i, 0)),
        out_shape=jax.ShapeDtypeStruct((N, D), jnp.float32),
    )(degp, degp, x, w)


def _mid_body(dega_ref, degb_ref, agg_ref, hs_ref, b_ref, w_ref, out_ref):
    dinv = _dinv_block(dega_ref[0], degb_ref[0])
    z = dinv * (agg_ref[0] + agg_ref[1] + hs_ref[...]) + b_ref[...]
    x2 = jnp.maximum(z, 0.0)
    h2 = jnp.dot(x2, w_ref[...], preferred_element_type=jnp.float32)
    out_ref[...] = h2 * dinv


def _tc_mid(degp, aggp, hs, b, w):
    return pl.pallas_call(
        _mid_body,
        grid=(N // BLK,),
        in_specs=_deg_specs() + [
            pl.BlockSpec((2, BLK, D), lambda i: (0, i, 0)),
            pl.BlockSpec((BLK, D), lambda i: (i, 0)),
            pl.BlockSpec((1, D), lambda i: (0, 0)),
            pl.BlockSpec((D, D), lambda i: (0, 0)),
        ],
        out_specs=pl.BlockSpec((BLK, D), lambda i: (i, 0)),
        out_shape=jax.ShapeDtypeStruct((N, D), jnp.float32),
    )(degp, degp, aggp, hs, b, w)


def _final_body(dega_ref, degb_ref, agg_ref, hs_ref, b_ref, out_ref):
    dinv = _dinv_block(dega_ref[0], degb_ref[0])
    z = dinv * (agg_ref[0] + agg_ref[1] + hs_ref[...]) + b_ref[...]
    m = jnp.max(z, axis=1, keepdims=True)
    zs = z - m
    lse = jnp.log(jnp.sum(jnp.exp(zs), axis=1, keepdims=True))
    out_ref[...] = zs - lse


def _tc_final(degp, aggp, hs, b):
    return pl.pallas_call(
        _final_body,
        grid=(N // BLK,),
        in_specs=_deg_specs() + [
            pl.BlockSpec((2, BLK, D), lambda i: (0, i, 0)),
            pl.BlockSpec((BLK, D), lambda i: (i, 0)),
            pl.BlockSpec((1, D), lambda i: (0, 0)),
        ],
        out_specs=pl.BlockSpec((BLK, D), lambda i: (i, 0)),
        out_shape=jax.ShapeDtypeStruct((N, D), jnp.float32),
    )(degp, degp, aggp, hs, b)


@jax.jit
def kernel(x, edge_index, W1, b1, W2, b2):
    ei = edge_index.astype(jnp.int32)
    src = ei[0]
    dst = ei[1]
    dst3 = dst.reshape(NW, NCHUNK, K)
    sd = jnp.stack([src.reshape(NW, NCHUNK, K), dst.reshape(NW, NCHUNK, K)], axis=2)
    b1r = b1.reshape(1, D)
    b2r = b2.reshape(1, D)
    zeros = jnp.zeros((NPAD, D), jnp.float32)
    ones = jnp.ones((K, DEGW), jnp.float32)

    degp = _deg_kernel(dst3, zeros, ones)
    hs1 = _tc_scale_matmul(degp, x, W1)
    agg1 = _agg_kernel(hs1, sd, zeros)
    hs2 = _tc_mid(degp, agg1, hs1, b1r, W2)
    agg2 = _agg_kernel(hs2, sd, zeros)
    return _tc_final(degp, agg2, hs2, b2r)


# speedup vs baseline: 1.0075x; 1.0075x over previous
import functools

import jax
import jax.numpy as jnp
from jax import lax
from jax.experimental import pallas as pl
from jax.experimental.pallas import tpu as pltpu
from jax.experimental.pallas import tpu_sc as plsc

N = 10000
NPAD = 10240
D = 128
E = 320000
NC = 2
NS = 16
L = 16
NW = NC * NS
EPW = E // NW
K = 40
NCHUNK = EPW // K
RPT = NPAD // NS
ZR = 32
DEGW = 128

_mesh = plsc.VectorSubcoreMesh(core_axis_name="c", subcore_axis_name="s")


NBUF = 5
GRP = NCHUNK // NBUF
ZRW = 640


@functools.partial(
    pl.kernel,
    out_type=jax.ShapeDtypeStruct((NC, NPAD, DEGW), jnp.float32),
    mesh=_mesh,
    scratch_types=[
        pltpu.VMEM_SHARED((NPAD, DEGW), jnp.float32),
        pltpu.VMEM((NCHUNK, K), jnp.int32),
        pltpu.VMEM((K, DEGW), jnp.float32),
        pltpu.SemaphoreType.DMA,
        pltpu.SemaphoreType.DMA,
        pltpu.SemaphoreType.DMA,
    ],
)
def _deg_kernel(dst_hbm, zeros_hbm, ones_hbm, out_hbm, acc, didx, ones_v, isem, zsem, ssem):
    cid = lax.axis_index("c")
    sid = lax.axis_index("s")
    wid = cid * NS + sid
    rbase = sid * RPT

    idx_cp = pltpu.async_copy(dst_hbm.at[wid], didx, isem)
    one_cp = pltpu.async_copy(ones_hbm, ones_v, isem)
    zero_cp = pltpu.async_copy(
        zeros_hbm.at[pl.ds(rbase, RPT), :], acc.at[pl.ds(rbase, RPT), :], zsem)
    idx_cp.wait()
    one_cp.wait()
    zero_cp.wait()
    plsc.subcore_barrier()

    @pl.loop(0, NCHUNK // 10)
    def _grp(g):
        cps = [
            pltpu.async_copy(ones_v, acc.at[didx.at[g * 10 + b]], ssem, add=True)
            for b in range(10)
        ]
        for cp in cps:
            cp.wait()

    plsc.subcore_barrier()

    @pl.loop(0, RPT // ZRW)
    def _write(r):
        row = rbase + r * ZRW
        pltpu.sync_copy(acc.at[pl.ds(row, ZRW), :], out_hbm.at[cid, pl.ds(row, ZRW), :])


ISLOT = 2 * NBUF


@functools.partial(
    pl.kernel,
    out_type=jax.ShapeDtypeStruct((NC, NPAD, D), jnp.float32),
    mesh=_mesh,
    scratch_types=[
        pltpu.VMEM_SHARED((NPAD, D), jnp.float32),
        [pltpu.VMEM((2, K), jnp.int32)] * ISLOT,
        [pltpu.VMEM((K, D), jnp.float32)] * NBUF,
        pltpu.SemaphoreType.DMA,
        [pltpu.SemaphoreType.DMA] * ISLOT,
        [pltpu.SemaphoreType.DMA] * NBUF,
    ],
)
def _agg_kernel(table_hbm, sd_hbm, zeros_hbm, out_hbm,
                acc, sdr, rows, zsem, isem, gsem):
    cid = lax.axis_index("c")
    sid = lax.axis_index("s")
    wid = cid * NS + sid
    rbase = sid * RPT

    zero_cp = pltpu.async_copy(
        zeros_hbm.at[pl.ds(rbase, RPT), :], acc.at[pl.ds(rbase, RPT), :], zsem)

    for s in range(ISLOT):
        pltpu.async_copy(sd_hbm.at[wid, s], sdr[s], isem[s])

    for u in range(NBUF):
        pltpu.make_async_copy(sd_hbm.at[0, 0], sdr[u], isem[u]).wait()
        pltpu.async_copy(table_hbm.at[sdr[u].at[0]], rows[u], gsem[u])

    zero_cp.wait()
    plsc.subcore_barrier()

    @pl.loop(0, NCHUNK // ISLOT)
    def _grp(g):
        for u in range(ISLOT):
            ci = g * ISLOT + u
            b = u % NBUF
            s2 = (u + NBUF) % ISLOT
            pltpu.make_async_copy(table_hbm.at[sdr[0].at[0]], rows[b], gsem[b]).wait()
            pltpu.sync_copy(rows[b], acc.at[sdr[u].at[1]], add=True)

            @pl.when(ci + ISLOT < NCHUNK)
            def _():
                pltpu.async_copy(sd_hbm.at[wid, ci + ISLOT], sdr[u], isem[u])

            @pl.when(ci + NBUF < NCHUNK)
            def _():
                pltpu.make_async_copy(sd_hbm.at[0, 0], sdr[s2], isem[s2]).wait()
                pltpu.async_copy(table_hbm.at[sdr[s2].at[0]], rows[b], gsem[b])

    plsc.subcore_barrier()

    @pl.loop(0, RPT // ZRW)
    def _write(r):
        row = rbase + r * ZRW
        pltpu.sync_copy(acc.at[pl.ds(row, ZRW), :], out_hbm.at[cid, pl.ds(row, ZRW), :])


BLK = 2000


def _dinv_block(dega, degb):
    deg = jnp.max(dega, axis=1, keepdims=True) + jnp.max(degb, axis=1, keepdims=True) + 1.0
    return lax.rsqrt(deg)


def _deg_specs():
    return [
        pl.BlockSpec((1, BLK, DEGW), lambda i: (0, i, 0)),
        pl.BlockSpec((1, BLK, DEGW), lambda i: (1, i, 0)),
    ]


def _scale_matmul_body(dega_ref, degb_ref, x_ref, w_ref, hs_ref):
    dinv = _dinv_block(dega_ref[0], degb_ref[0])
    h = jnp.dot(x_ref[...], w_ref[...], preferred_element_type=jnp.float32)
    hs_ref[...] = h * dinv


def _tc_scale_matmul(degp, x, w):
    return pl.pallas_call(
        _scale_matmul_body,
        grid=(N // BLK,),
        in_specs=_deg_specs() + [
            pl.BlockSpec((BLK, D), lambda i: (i, 0)),
            pl.BlockSpec((D, D), lambda i: (0, 0)),
        ],
        out_specs=pl.BlockSpec((BLK, D), lambda i: (i, 0)),
        out_shape=jax.ShapeDtypeStruct((N, D), jnp.float32),
    )(degp, degp, x, w)


def _mid_body(dega_ref, degb_ref, agg_ref, hs_ref, b_ref, w_ref, out_ref):
    dinv = _dinv_block(dega_ref[0], degb_ref[0])
    z = dinv * (agg_ref[0] + agg_ref[1] + hs_ref[...]) + b_ref[...]
    x2 = jnp.maximum(z, 0.0)
    h2 = jnp.dot(x2, w_ref[...], preferred_element_type=jnp.float32)
    out_ref[...] = h2 * dinv


def _tc_mid(degp, aggp, hs, b, w):
    return pl.pallas_call(
        _mid_body,
        grid=(N // BLK,),
        in_specs=_deg_specs() + [
            pl.BlockSpec((2, BLK, D), lambda i: (0, i, 0)),
            pl.BlockSpec((BLK, D), lambda i: (i, 0)),
            pl.BlockSpec((1, D), lambda i: (0, 0)),
            pl.BlockSpec((D, D), lambda i: (0, 0)),
        ],
        out_specs=pl.BlockSpec((BLK, D), lambda i: (i, 0)),
        out_shape=jax.ShapeDtypeStruct((N, D), jnp.float32),
    )(degp, degp, aggp, hs, b, w)


def _final_body(dega_ref, degb_ref, agg_ref, hs_ref, b_ref, out_ref):
    dinv = _dinv_block(dega_ref[0], degb_ref[0])
    z = dinv * (agg_ref[0] + agg_ref[1] + hs_ref[...]) + b_ref[...]
    m = jnp.max(z, axis=1, keepdims=True)
    zs = z - m
    lse = jnp.log(jnp.sum(jnp.exp(zs), axis=1, keepdims=True))
    out_ref[...] = zs - lse


def _tc_final(degp, aggp, hs, b):
    return pl.pallas_call(
        _final_body,
        grid=(N // BLK,),
        in_specs=_deg_specs() + [
            pl.BlockSpec((2, BLK, D), lambda i: (0, i, 0)),
            pl.BlockSpec((BLK, D), lambda i: (i, 0)),
            pl.BlockSpec((1, D), lambda i: (0, 0)),
        ],
        out_specs=pl.BlockSpec((BLK, D), lambda i: (i, 0)),
        out_shape=jax.ShapeDtypeStruct((N, D), jnp.float32),
    )(degp, degp, aggp, hs, b)


@jax.jit
def kernel(x, edge_index, W1, b1, W2, b2):
    ei = edge_index.astype(jnp.int32)
    src = ei[0]
    dst = ei[1]
    dst3 = dst.reshape(NW, NCHUNK, K)
    sd = jnp.stack([src.reshape(NW, NCHUNK, K), dst.reshape(NW, NCHUNK, K)], axis=2)
    b1r = b1.reshape(1, D)
    b2r = b2.reshape(1, D)
    zeros = jnp.zeros((NPAD, D), jnp.float32)
    ones = jnp.ones((K, DEGW), jnp.float32)

    degp = _deg_kernel(dst3, zeros, ones)
    hs1 = _tc_scale_matmul(degp, x, W1)
    agg1 = _agg_kernel(hs1, sd, zeros)
    hs2 = _tc_mid(degp, agg1, hs1, b1r, W2)
    agg2 = _agg_kernel(hs2, sd, zeros)
    return _tc_final(degp, agg2, hs2, b2r)


# speedup vs baseline: 1.0076x; 1.0000x over previous
import functools

import jax
import jax.numpy as jnp
from jax import lax
from jax.experimental import pallas as pl
from jax.experimental.pallas import tpu as pltpu
from jax.experimental.pallas import tpu_sc as plsc

N = 10000
NPAD = 10240
D = 128
E = 320000
NC = 2
NS = 16
L = 16
NW = NC * NS
EPW = E // NW
K = 40
NCHUNK = EPW // K
RPT = NPAD // NS
DEGW = 128

_mesh = plsc.VectorSubcoreMesh(core_axis_name="c", subcore_axis_name="s")


NBUF = 5
ZRW = 640


@functools.partial(
    pl.kernel,
    out_type=jax.ShapeDtypeStruct((NC, NPAD, DEGW), jnp.float32),
    mesh=_mesh,
    scratch_types=[
        pltpu.VMEM_SHARED((NPAD, DEGW), jnp.float32),
        pltpu.VMEM((NCHUNK, K), jnp.int32),
        pltpu.VMEM((K, DEGW), jnp.float32),
        pltpu.SemaphoreType.DMA,
        pltpu.SemaphoreType.DMA,
        pltpu.SemaphoreType.DMA,
    ],
)
def _deg_kernel(dst_hbm, zeros_hbm, ones_hbm, out_hbm, acc, didx, ones_v, isem, zsem, ssem):
    cid = lax.axis_index("c")
    sid = lax.axis_index("s")
    wid = cid * NS + sid
    rbase = sid * RPT

    idx_cp = pltpu.async_copy(dst_hbm.at[wid], didx, isem)
    one_cp = pltpu.async_copy(ones_hbm, ones_v, isem)
    zero_cp = pltpu.async_copy(
        zeros_hbm.at[pl.ds(rbase, RPT), :], acc.at[pl.ds(rbase, RPT), :], zsem)
    idx_cp.wait()
    one_cp.wait()
    zero_cp.wait()
    plsc.subcore_barrier()

    @pl.loop(0, NCHUNK // 10)
    def _grp(g):
        cps = [
            pltpu.async_copy(ones_v, acc.at[didx.at[g * 10 + b]], ssem, add=True)
            for b in range(10)
        ]
        for cp in cps:
            cp.wait()

    plsc.subcore_barrier()

    @pl.loop(0, RPT // ZRW)
    def _write(r):
        row = rbase + r * ZRW
        pltpu.sync_copy(acc.at[pl.ds(row, ZRW), :], out_hbm.at[cid, pl.ds(row, ZRW), :])


ISLOT = 2 * NBUF


@functools.partial(
    pl.kernel,
    out_type=jax.ShapeDtypeStruct((NC, NPAD, D), jnp.float32),
    mesh=_mesh,
    scratch_types=[
        pltpu.VMEM_SHARED((NPAD, D), jnp.float32),
        [pltpu.VMEM((2, K), jnp.int32)] * ISLOT,
        [pltpu.VMEM((K, D), jnp.float32)] * NBUF,
        pltpu.SemaphoreType.DMA,
        [pltpu.SemaphoreType.DMA] * ISLOT,
        [pltpu.SemaphoreType.DMA] * NBUF,
    ],
)
def _agg_kernel(table_hbm, sd_hbm, zeros_hbm, out_hbm,
                acc, sdr, rows, zsem, isem, gsem):
    cid = lax.axis_index("c")
    sid = lax.axis_index("s")
    wid = cid * NS + sid
    rbase = sid * RPT

    zero_cp = pltpu.async_copy(
        zeros_hbm.at[pl.ds(rbase, RPT), :], acc.at[pl.ds(rbase, RPT), :], zsem)

    for s in range(ISLOT):
        pltpu.async_copy(sd_hbm.at[wid, s], sdr[s], isem[s])

    for u in range(NBUF):
        pltpu.make_async_copy(sd_hbm.at[0, 0], sdr[u], isem[u]).wait()
        pltpu.async_copy(table_hbm.at[sdr[u].at[0]], rows[u], gsem[u])

    zero_cp.wait()
    plsc.subcore_barrier()

    @pl.loop(0, NCHUNK // ISLOT)
    def _grp(g):
        for u in range(ISLOT):
            ci = g * ISLOT + u
            b = u % NBUF
            s2 = (u + NBUF) % ISLOT
            pltpu.make_async_copy(table_hbm.at[sdr[0].at[0]], rows[b], gsem[b]).wait()
            pltpu.sync_copy(rows[b], acc.at[sdr[u].at[1]], add=True)

            @pl.when(ci + ISLOT < NCHUNK)
            def _():
                pltpu.async_copy(sd_hbm.at[wid, ci + ISLOT], sdr[u], isem[u])

            @pl.when(ci + NBUF < NCHUNK)
            def _():
                pltpu.make_async_copy(sd_hbm.at[0, 0], sdr[s2], isem[s2]).wait()
                pltpu.async_copy(table_hbm.at[sdr[s2].at[0]], rows[b], gsem[b])

    plsc.subcore_barrier()

    @pl.loop(0, RPT // ZRW)
    def _write(r):
        row = rbase + r * ZRW
        pltpu.sync_copy(acc.at[pl.ds(row, ZRW), :], out_hbm.at[cid, pl.ds(row, ZRW), :])


BLK = 2000


def _dinv_block(dega, degb):
    deg = jnp.max(dega, axis=1, keepdims=True) + jnp.max(degb, axis=1, keepdims=True) + 1.0
    return lax.rsqrt(deg)


def _deg_specs():
    return [
        pl.BlockSpec((1, BLK, DEGW), lambda i: (0, i, 0)),
        pl.BlockSpec((1, BLK, DEGW), lambda i: (1, i, 0)),
    ]


def _scale_matmul_body(dega_ref, degb_ref, x_ref, w_ref, hs_ref):
    dinv = _dinv_block(dega_ref[0], degb_ref[0])
    h = jnp.dot(x_ref[...], w_ref[...], preferred_element_type=jnp.float32)
    hs_ref[...] = h * dinv


def _tc_scale_matmul(degp, x, w):
    return pl.pallas_call(
        _scale_matmul_body,
        grid=(N // BLK,),
        in_specs=_deg_specs() + [
            pl.BlockSpec((BLK, D), lambda i: (i, 0)),
            pl.BlockSpec((D, D), lambda i: (0, 0)),
        ],
        out_specs=pl.BlockSpec((BLK, D), lambda i: (i, 0)),
        out_shape=jax.ShapeDtypeStruct((N, D), jnp.float32),
    )(degp, degp, x, w)


def _mid_body(dega_ref, degb_ref, agg_ref, hs_ref, b_ref, w_ref, out_ref):
    dinv = _dinv_block(dega_ref[0], degb_ref[0])
    z = dinv * (agg_ref[0] + agg_ref[1] + hs_ref[...]) + b_ref[...]
    x2 = jnp.maximum(z, 0.0)
    h2 = jnp.dot(x2, w_ref[...], preferred_element_type=jnp.float32)
    out_ref[...] = h2 * dinv


def _tc_mid(degp, aggp, hs, b, w):
    return pl.pallas_call(
        _mid_body,
        grid=(N // BLK,),
        in_specs=_deg_specs() + [
            pl.BlockSpec((2, BLK, D), lambda i: (0, i, 0)),
            pl.BlockSpec((BLK, D), lambda i: (i, 0)),
            pl.BlockSpec((1, D), lambda i: (0, 0)),
            pl.BlockSpec((D, D), lambda i: (0, 0)),
        ],
        out_specs=pl.BlockSpec((BLK, D), lambda i: (i, 0)),
        out_shape=jax.ShapeDtypeStruct((N, D), jnp.float32),
    )(degp, degp, aggp, hs, b, w)


def _final_body(dega_ref, degb_ref, agg_ref, hs_ref, b_ref, out_ref):
    dinv = _dinv_block(dega_ref[0], degb_ref[0])
    z = dinv * (agg_ref[0] + agg_ref[1] + hs_ref[...]) + b_ref[...]
    m = jnp.max(z, axis=1, keepdims=True)
    zs = z - m
    lse = jnp.log(jnp.sum(jnp.exp(zs), axis=1, keepdims=True))
    out_ref[...] = zs - lse


def _tc_final(degp, aggp, hs, b):
    return pl.pallas_call(
        _final_body,
        grid=(N // BLK,),
        in_specs=_deg_specs() + [
            pl.BlockSpec((2, BLK, D), lambda i: (0, i, 0)),
            pl.BlockSpec((BLK, D), lambda i: (i, 0)),
            pl.BlockSpec((1, D), lambda i: (0, 0)),
        ],
        out_specs=pl.BlockSpec((BLK, D), lambda i: (i, 0)),
        out_shape=jax.ShapeDtypeStruct((N, D), jnp.float32),
    )(degp, degp, aggp, hs, b)


@jax.jit
def kernel(x, edge_index, W1, b1, W2, b2):
    ei = edge_index.astype(jnp.int32)
    src = ei[0]
    dst = ei[1]
    dst3 = dst.reshape(NW, NCHUNK, K)
    sd = jnp.stack([src.reshape(NW, NCHUNK, K), dst.reshape(NW, NCHUNK, K)], axis=2)
    b1r = b1.reshape(1, D)
    b2r = b2.reshape(1, D)
    zeros = jnp.zeros((NPAD, D), jnp.float32)
    ones = jnp.ones((K, DEGW), jnp.float32)

    degp = _deg_kernel(dst3, zeros, ones)
    hs1 = _tc_scale_matmul(degp, x, W1)
    agg1 = _agg_kernel(hs1, sd, zeros)
    hs2 = _tc_mid(degp, agg1, hs1, b1r, W2)
    agg2 = _agg_kernel(hs2, sd, zeros)
    return _tc_final(degp, agg2, hs2, b2r)


# speedup vs baseline: 1.0076x; 1.0000x over previous
import functools

import jax
import jax.numpy as jnp
from jax import lax
from jax.experimental import pallas as pl
from jax.experimental.pallas import tpu as pltpu
from jax.experimental.pallas import tpu_sc as plsc

N = 10000
NPAD = 10240
D = 128
E = 320000
NC = 2
NS = 16
L = 16
NW = NC * NS
EPW = E // NW
K = 40
NCHUNK = EPW // K
RPT = NPAD // NS
DEGW = 128

_mesh = plsc.VectorSubcoreMesh(core_axis_name="c", subcore_axis_name="s")


NBUF = 5
ZRW = 640


@functools.partial(
    pl.kernel,
    out_type=jax.ShapeDtypeStruct((NC, NPAD, DEGW), jnp.float32),
    mesh=_mesh,
    scratch_types=[
        pltpu.VMEM_SHARED((NPAD, DEGW), jnp.float32),
        pltpu.VMEM((NCHUNK, K), jnp.int32),
        pltpu.VMEM((K, DEGW), jnp.float32),
        pltpu.SemaphoreType.DMA,
        pltpu.SemaphoreType.DMA,
        pltpu.SemaphoreType.DMA,
    ],
)
def _deg_kernel(dst_hbm, zeros_hbm, ones_hbm, out_hbm, acc, didx, ones_v, isem, zsem, ssem):
    cid = lax.axis_index("c")
    sid = lax.axis_index("s")
    wid = cid * NS + sid
    rbase = sid * RPT

    idx_cp = pltpu.async_copy(dst_hbm.at[wid], didx, isem)
    one_cp = pltpu.async_copy(ones_hbm, ones_v, isem)
    zero_cp = pltpu.async_copy(
        zeros_hbm.at[pl.ds(rbase, RPT), :], acc.at[pl.ds(rbase, RPT), :], zsem)
    idx_cp.wait()
    one_cp.wait()
    zero_cp.wait()
    plsc.subcore_barrier()

    @pl.loop(0, NCHUNK // 10)
    def _grp(g):
        cps = [
            pltpu.async_copy(ones_v, acc.at[didx.at[g * 10 + b]], ssem, add=True)
            for b in range(10)
        ]
        for cp in cps:
            cp.wait()

    plsc.subcore_barrier()

    @pl.loop(0, RPT // ZRW)
    def _write(r):
        row = rbase + r * ZRW
        pltpu.sync_copy(acc.at[pl.ds(row, ZRW), :], out_hbm.at[cid, pl.ds(row, ZRW), :])


ISLOT = 2 * NBUF


@functools.partial(
    pl.kernel,
    out_type=jax.ShapeDtypeStruct((NC, NPAD, D), jnp.float32),
    mesh=_mesh,
    scratch_types=[
        pltpu.VMEM_SHARED((NPAD, D), jnp.float32),
        [pltpu.VMEM((2, K), jnp.int32)] * ISLOT,
        [pltpu.VMEM((K, D), jnp.float32)] * NBUF,
        pltpu.SemaphoreType.DMA,
        [pltpu.SemaphoreType.DMA] * ISLOT,
        [pltpu.SemaphoreType.DMA] * NBUF,
    ],
)
def _agg_kernel(table_hbm, sd_hbm, zeros_hbm, out_hbm,
                acc, sdr, rows, zsem, isem, gsem):
    cid = lax.axis_index("c")
    sid = lax.axis_index("s")
    wid = cid * NS + sid
    rbase = sid * RPT

    zero_cp = pltpu.async_copy(
        zeros_hbm.at[pl.ds(rbase, RPT), :], acc.at[pl.ds(rbase, RPT), :], zsem)

    for s in range(ISLOT):
        pltpu.async_copy(sd_hbm.at[wid, s], sdr[s], isem[s])

    for u in range(NBUF):
        pltpu.make_async_copy(sd_hbm.at[0, 0], sdr[u], isem[u]).wait()
        pltpu.async_copy(table_hbm.at[sdr[u].at[0]], rows[u], gsem[u])

    zero_cp.wait()
    plsc.subcore_barrier()

    @pl.loop(0, NCHUNK // ISLOT)
    def _grp(g):
        for u in range(ISLOT):
            ci = g * ISLOT + u
            b = u % NBUF
            s2 = (u + NBUF) % ISLOT
            pltpu.make_async_copy(table_hbm.at[sdr[0].at[0]], rows[b], gsem[b]).wait()
            pltpu.sync_copy(rows[b], acc.at[sdr[u].at[1]], add=True)

            @pl.when(ci + ISLOT < NCHUNK)
            def _():
                pltpu.async_copy(sd_hbm.at[wid, ci + ISLOT], sdr[u], isem[u])

            @pl.when(ci + NBUF < NCHUNK)
            def _():
                pltpu.make_async_copy(sd_hbm.at[0, 0], sdr[s2], isem[s2]).wait()
                pltpu.async_copy(table_hbm.at[sdr[s2].at[0]], rows[b], gsem[b])

    plsc.subcore_barrier()

    @pl.loop(0, RPT // ZRW)
    def _write(r):
        row = rbase + r * ZRW
        pltpu.sync_copy(acc.at[pl.ds(row, ZRW), :], out_hbm.at[cid, pl.ds(row, ZRW), :])


BLK = 2000
_TC_PARAMS = pltpu.CompilerParams(dimension_semantics=("parallel",))


def _dinv_block(dega, degb):
    deg = jnp.max(dega, axis=1, keepdims=True) + jnp.max(degb, axis=1, keepdims=True) + 1.0
    return lax.rsqrt(deg)


def _deg_specs():
    return [
        pl.BlockSpec((1, BLK, DEGW), lambda i: (0, i, 0)),
        pl.BlockSpec((1, BLK, DEGW), lambda i: (1, i, 0)),
    ]


def _scale_matmul_body(dega_ref, degb_ref, x_ref, w_ref, hs_ref):
    dinv = _dinv_block(dega_ref[0], degb_ref[0])
    h = jnp.dot(x_ref[...], w_ref[...], preferred_element_type=jnp.float32)
    hs_ref[...] = h * dinv


def _tc_scale_matmul(degp, x, w):
    return pl.pallas_call(
        _scale_matmul_body,
        grid=(N // BLK,),
        compiler_params=_TC_PARAMS,
        in_specs=_deg_specs() + [
            pl.BlockSpec((BLK, D), lambda i: (i, 0)),
            pl.BlockSpec((D, D), lambda i: (0, 0)),
        ],
        out_specs=pl.BlockSpec((BLK, D), lambda i: (i, 0)),
        out_shape=jax.ShapeDtypeStruct((N, D), jnp.float32),
    )(degp, degp, x, w)


def _mid_body(dega_ref, degb_ref, agg_ref, hs_ref, b_ref, w_ref, out_ref):
    dinv = _dinv_block(dega_ref[0], degb_ref[0])
    z = dinv * (agg_ref[0] + agg_ref[1] + hs_ref[...]) + b_ref[...]
    x2 = jnp.maximum(z, 0.0)
    h2 = jnp.dot(x2, w_ref[...], preferred_element_type=jnp.float32)
    out_ref[...] = h2 * dinv


def _tc_mid(degp, aggp, hs, b, w):
    return pl.pallas_call(
        _mid_body,
        grid=(N // BLK,),
        compiler_params=_TC_PARAMS,
        in_specs=_deg_specs() + [
            pl.BlockSpec((2, BLK, D), lambda i: (0, i, 0)),
            pl.BlockSpec((BLK, D), lambda i: (i, 0)),
            pl.BlockSpec((1, D), lambda i: (0, 0)),
            pl.BlockSpec((D, D), lambda i: (0, 0)),
        ],
        out_specs=pl.BlockSpec((BLK, D), lambda i: (i, 0)),
        out_shape=jax.ShapeDtypeStruct((N, D), jnp.float32),
    )(degp, degp, aggp, hs, b, w)


def _final_body(dega_ref, degb_ref, agg_ref, hs_ref, b_ref, out_ref):
    dinv = _dinv_block(dega_ref[0], degb_ref[0])
    z = dinv * (agg_ref[0] + agg_ref[1] + hs_ref[...]) + b_ref[...]
    m = jnp.max(z, axis=1, keepdims=True)
    zs = z - m
    lse = jnp.log(jnp.sum(jnp.exp(zs), axis=1, keepdims=True))
    out_ref[...] = zs - lse


def _tc_final(degp, aggp, hs, b):
    return pl.pallas_call(
        _final_body,
        grid=(N // BLK,),
        compiler_params=_TC_PARAMS,
        in_specs=_deg_specs() + [
            pl.BlockSpec((2, BLK, D), lambda i: (0, i, 0)),
            pl.BlockSpec((BLK, D), lambda i: (i, 0)),
            pl.BlockSpec((1, D), lambda i: (0, 0)),
        ],
        out_specs=pl.BlockSpec((BLK, D), lambda i: (i, 0)),
        out_shape=jax.ShapeDtypeStruct((N, D), jnp.float32),
    )(degp, degp, aggp, hs, b)


@jax.jit
def kernel(x, edge_index, W1, b1, W2, b2):
    ei = edge_index.astype(jnp.int32)
    src = ei[0]
    dst = ei[1]
    dst3 = dst.reshape(NW, NCHUNK, K)
    sd = jnp.stack([src.reshape(NW, NCHUNK, K), dst.reshape(NW, NCHUNK, K)], axis=2)
    b1r = b1.reshape(1, D)
    b2r = b2.reshape(1, D)
    zeros = jnp.zeros((NPAD, D), jnp.float32)
    ones = jnp.ones((K, DEGW), jnp.float32)

    degp = _deg_kernel(dst3, zeros, ones)
    hs1 = _tc_scale_matmul(degp, x, W1)
    agg1 = _agg_kernel(hs1, sd, zeros)
    hs2 = _tc_mid(degp, agg1, hs1, b1r, W2)
    agg2 = _agg_kernel(hs2, sd, zeros)
    return _tc_final(degp, agg2, hs2, b2r)
